# baseline (speedup 1.0000x reference)
import jax
import jax.numpy as jnp
from jax.experimental import pallas as pl
from jax.experimental.pallas import tpu as pltpu
from jax.experimental.pallas import tpu_sc as plsc

_W = 640
_BLK = 8192


def _pack_block(t_ref, o_ref, scr_a, scr_b):
    for c in range(_BLK // 128):
        scr = scr_a if c % 2 == 0 else scr_b
        scr[:, 0:64] = t_ref[:, pl.ds(128 * c, 128)].T
        o_ref[pl.ds(64 * c, 64), 0:64] = scr[pl.ds(0, 64, 2), 0:64]
        o_ref[pl.ds(64 * c, 64), 64:128] = scr[pl.ds(1, 64, 2), 0:64]


def _make_packed(table_t):
    V = table_t.shape[1]
    grid = (V + _BLK - 1) // _BLK

    return pl.pallas_call(
        _pack_block,
        grid=(grid,),
        in_specs=[pl.BlockSpec((64, _BLK), lambda i: (0, i))],
        out_specs=pl.BlockSpec((_BLK // 2, 128), lambda i: (i, 0)),
        out_shape=jax.ShapeDtypeStruct((V // 2, 128), jnp.float32),
        scratch_shapes=[pltpu.VMEM((128, 128), jnp.float32),
                        pltpu.VMEM((128, 128), jnp.float32)],
        compiler_params=pltpu.CompilerParams(
            dimension_semantics=("parallel",)),
    )(table_t)


def kernel(x, table):
    B, S = x.shape
    n = B * S
    V, D = table.shape
    idx = x.reshape(1, n)

    tt = table.T
    copied = pl.pallas_call(
        lambda t_ref, o_ref: o_ref.__setitem__((...,), t_ref[...]),
        grid=(tt.shape[1] // 8192,),
        in_specs=[pl.BlockSpec((64, 8192), lambda i: (0, i))],
        out_specs=pl.BlockSpec((64, 8192), lambda i: (0, i)),
        out_shape=jax.ShapeDtypeStruct(tt.shape, jnp.float32),
    )(tt)
    return copied[:, :n].T.reshape(B, S, D)

    mesh = plsc.VectorSubcoreMesh(core_axis_name="core",
                                  subcore_axis_name="subcore")

    @pl.kernel(out_type=jax.ShapeDtypeStruct((n, D), table.dtype), mesh=mesh,
               compiler_params=pltpu.CompilerParams(use_tc_tiling_on_sc=False))
    def gather_kernel(table_hbm, idx_hbm, out_hbm):
        def body(idx_vmem, out_vmem):
            pltpu.sync_copy(table_hbm.at[idx_vmem.at[0]], out_vmem)

        pltpu.emit_pipeline(
            body,
            grid=(n // _W,),
            in_specs=[pl.BlockSpec((1, _W), index_map=lambda i: (0, i))],
            out_specs=[pl.BlockSpec((_W, D), index_map=lambda i: (i, 0))],
            core_axis_name=("core", "subcore"),
            dimension_semantics=(pltpu.PARALLEL,),
        )(idx_hbm, out_hbm)

    out = gather_kernel(table_rm, idx)
    return out.reshape(B, S, D)


# speedup vs baseline: 1.1618x; 1.1618x over previous
import jax
import jax.numpy as jnp
from jax.experimental import pallas as pl
from jax.experimental.pallas import tpu as pltpu
from jax.experimental.pallas import tpu_sc as plsc

_W = 640
_BLK = 16384


def _pack_block(t_ref, o_ref, scr_a, scr_b):
    for c in range(_BLK // 128):
        scr = scr_a if c % 2 == 0 else scr_b
        scr[:, 0:64] = t_ref[:, pl.ds(128 * c, 128)].T
        o_ref[pl.ds(64 * c, 64), 0:64] = scr[pl.ds(0, 64, 2), 0:64]
        o_ref[pl.ds(64 * c, 64), 64:128] = scr[pl.ds(1, 64, 2), 0:64]


def _make_packed(table_t):
    V = table_t.shape[1]
    grid = (V + _BLK - 1) // _BLK

    return pl.pallas_call(
        _pack_block,
        grid=(grid,),
        in_specs=[pl.BlockSpec((64, _BLK), lambda i: (0, i))],
        out_specs=pl.BlockSpec((_BLK // 2, 128), lambda i: (i, 0)),
        out_shape=jax.ShapeDtypeStruct((V // 2, 128), jnp.float32),
        scratch_shapes=[pltpu.VMEM((128, 128), jnp.float32),
                        pltpu.VMEM((128, 128), jnp.float32)],
        compiler_params=pltpu.CompilerParams(
            dimension_semantics=("parallel",)),
    )(table_t)


def kernel(x, table):
    B, S = x.shape
    n = B * S
    V, D = table.shape
    idx = x.reshape(1, n)

    table_rm = _make_packed(table.T).reshape(V, D)

    mesh = plsc.VectorSubcoreMesh(core_axis_name="core",
                                  subcore_axis_name="subcore")

    @pl.kernel(out_type=jax.ShapeDtypeStruct((n, D), table.dtype), mesh=mesh,
               compiler_params=pltpu.CompilerParams(use_tc_tiling_on_sc=False))
    def gather_kernel(table_hbm, idx_hbm, out_hbm):
        def body(idx_vmem, out_vmem):
            pltpu.sync_copy(table_hbm.at[idx_vmem.at[0]], out_vmem)

        pltpu.emit_pipeline(
            body,
            grid=(n // _W,),
            in_specs=[pl.BlockSpec((1, _W), index_map=lambda i: (0, i))],
            out_specs=[pl.BlockSpec((_W, D), index_map=lambda i: (i, 0))],
            core_axis_name=("core", "subcore"),
            dimension_semantics=(pltpu.PARALLEL,),
        )(idx_hbm, out_hbm)

    out = gather_kernel(table_rm, idx)
    return out.reshape(B, S, D)


# speedup vs baseline: 1.1651x; 1.0028x over previous
import jax
import jax.numpy as jnp
from jax.experimental import pallas as pl
from jax.experimental.pallas import tpu as pltpu
from jax.experimental.pallas import tpu_sc as plsc

_W = 640
_BLK = 32768


def _pack_block(t_ref, o_ref, scr_a, scr_b):
    for c in range(_BLK // 128):
        scr = scr_a if c % 2 == 0 else scr_b
        scr[:, 0:64] = t_ref[:, pl.ds(128 * c, 128)].T
        o_ref[pl.ds(64 * c, 64), 0:64] = scr[pl.ds(0, 64, 2), 0:64]
        o_ref[pl.ds(64 * c, 64), 64:128] = scr[pl.ds(1, 64, 2), 0:64]


def _make_packed(table_t):
    V = table_t.shape[1]
    grid = (V + _BLK - 1) // _BLK

    return pl.pallas_call(
        _pack_block,
        grid=(grid,),
        in_specs=[pl.BlockSpec((64, _BLK), lambda i: (0, i))],
        out_specs=pl.BlockSpec((_BLK // 2, 128), lambda i: (i, 0)),
        out_shape=jax.ShapeDtypeStruct((V // 2, 128), jnp.float32),
        scratch_shapes=[pltpu.VMEM((128, 128), jnp.float32),
                        pltpu.VMEM((128, 128), jnp.float32)],
        compiler_params=pltpu.CompilerParams(
            dimension_semantics=("parallel",)),
    )(table_t)


def kernel(x, table):
    B, S = x.shape
    n = B * S
    V, D = table.shape
    idx = x.reshape(1, n)

    table_rm = _make_packed(table.T).reshape(V, D)

    mesh = plsc.VectorSubcoreMesh(core_axis_name="core",
                                  subcore_axis_name="subcore")

    @pl.kernel(out_type=jax.ShapeDtypeStruct((n, D), table.dtype), mesh=mesh,
               compiler_params=pltpu.CompilerParams(use_tc_tiling_on_sc=False))
    def gather_kernel(table_hbm, idx_hbm, out_hbm):
        def body(idx_vmem, out_vmem):
            pltpu.sync_copy(table_hbm.at[idx_vmem.at[0]], out_vmem)

        pltpu.emit_pipeline(
            body,
            grid=(n // _W,),
            in_specs=[pl.BlockSpec((1, _W), index_map=lambda i: (0, i))],
            out_specs=[pl.BlockSpec((_W, D), index_map=lambda i: (i, 0))],
            core_axis_name=("core", "subcore"),
            dimension_semantics=(pltpu.PARALLEL,),
        )(idx_hbm, out_hbm)

    out = gather_kernel(table_rm, idx)
    return out.reshape(B, S, D)
